# baseline (speedup 1.0000x reference)
import jax
import jax.numpy as jnp
import numpy as np
from jax.experimental import pallas as pl
from jax.experimental.pallas import tpu as pltpu

N_PTS = 8192
TILE = 512
NTILE = N_PTS // TILE
NSRC = 64
K1 = 8
NC = 16
DIM = 3
IN_CH = 3
CIN = 192
COUT = 96


def _gmat_kernel(ft_ref, wct_ref, g_ref):
    for c in range(NC):
        g_ref[c] = jnp.dot(wct_ref[c], ft_ref[...],
                           precision=jax.lax.Precision.HIGHEST, preferred_element_type=jnp.float32)


def _cv1d_kernel(qt_ref, p_ref, pt_ref, at_ref, b1_ref, l2t_ref, b2_ref,
                 l3t_ref, b3_ref, g_ref, y_ref, s1_ref, s2_ref):
    qx = qt_ref[0:1, :]
    qy = qt_ref[1:2, :]
    qz = qt_ref[2:3, :]
    px = p_ref[:, 0:1]
    py = p_ref[:, 1:2]
    pz = p_ref[:, 2:3]
    dx = px - qx
    dy = py - qy
    dz = pz - qz
    d2 = dx * dx + dy * dy + dz * dz

    at = at_ref[...]
    pa = jnp.dot(at, pt_ref[...],
                 precision=jax.lax.Precision.HIGHEST, preferred_element_type=jnp.float32)
    qa = jnp.dot(at, qt_ref[...],
                 precision=jax.lax.Precision.HIGHEST, preferred_element_type=jnp.float32)

    iota = jax.lax.broadcasted_iota(jnp.int32, (NSRC, TILE), 0)
    d2w = d2
    nrm_acc = None
    ohs = []
    relas = []
    for _ in range(K1):
        idx = jnp.argmin(d2w, axis=0, keepdims=True)
        msk = iota == idx
        oh = msk.astype(jnp.float32)
        n2 = jnp.min(d2w, axis=0, keepdims=True)
        rela = jnp.dot(pa, oh,
                       precision=jax.lax.Precision.HIGHEST, preferred_element_type=jnp.float32) - qa
        nrm = jnp.sqrt(n2 + 1e-9)
        nrm_acc = nrm if nrm_acc is None else nrm_acc + nrm
        ohs.append(oh)
        relas.append(rela)
        d2w = jnp.where(msk, jnp.inf, d2w)

    inv = 1.0 / (nrm_acc / K1 + 1e-6)
    b1 = b1_ref[...]
    l2t = l2t_ref[...]
    b2 = b2_ref[...]
    l3t = l3t_ref[...]
    b3 = b3_ref[...]

    h3s = []
    for k in range(K1):
        h = jnp.maximum(relas[k] * inv + b1, 0.0)
        h = jnp.maximum(
            jnp.dot(l2t, h, precision=jax.lax.Precision.HIGHEST, preferred_element_type=jnp.float32) + b2, 0.0)
        h = jnp.dot(l3t, h, precision=jax.lax.Precision.HIGHEST, preferred_element_type=jnp.float32) + b3
        h3s.append(h)

    acc = jnp.zeros((COUT, TILE), dtype=jnp.float32)
    for c in range(NC):
        sc = h3s[0][c:c + 1, :] * ohs[0]
        for k in range(1, K1):
            sc = sc + h3s[k][c:c + 1, :] * ohs[k]
        acc = acc + jnp.dot(g_ref[c], sc,
                            precision=jax.lax.Precision.HIGHEST, preferred_element_type=jnp.float32)
    y = acc / K1
    y_ref[...] = y
    s1_ref[...] = jnp.sum(y, axis=1, keepdims=True)[None]
    s2_ref[...] = jnp.sum(y * y, axis=1, keepdims=True)[None]


def _run_cv1d(qpts_t, pts1_n, pts1_t, feat_src_t, p):
    l1w = p['l1_w']
    a_t = l1w.reshape(NC, DIM, 2 * NC).sum(axis=0).T
    cvec = p['centers'].reshape(1, NC * DIM) @ l1w
    b1p = (p['l1_b'][None, :] - cvec).T
    wct = jnp.transpose(p['weight'].reshape(NC, CIN, COUT), (0, 2, 1))

    gmat = pl.pallas_call(
        _gmat_kernel,
        in_specs=[
            pl.BlockSpec((CIN, NSRC), lambda: (0, 0)),
            pl.BlockSpec((NC, COUT, CIN), lambda: (0, 0, 0)),
        ],
        out_specs=pl.BlockSpec((NC, COUT, NSRC), lambda: (0, 0, 0)),
        out_shape=jax.ShapeDtypeStruct((NC, COUT, NSRC), jnp.float32),
    )(feat_src_t, wct)

    return pl.pallas_call(
        _cv1d_kernel,
        grid=(NTILE,),
        in_specs=[
            pl.BlockSpec((3, TILE), lambda i: (0, i)),
            pl.BlockSpec((NSRC, 3), lambda i: (0, 0)),
            pl.BlockSpec((3, NSRC), lambda i: (0, 0)),
            pl.BlockSpec((2 * NC, 3), lambda i: (0, 0)),
            pl.BlockSpec((2 * NC, 1), lambda i: (0, 0)),
            pl.BlockSpec((NC, 2 * NC), lambda i: (0, 0)),
            pl.BlockSpec((NC, 1), lambda i: (0, 0)),
            pl.BlockSpec((NC, NC), lambda i: (0, 0)),
            pl.BlockSpec((NC, 1), lambda i: (0, 0)),
            pl.BlockSpec((NC, COUT, NSRC), lambda i: (0, 0, 0)),
        ],
        out_specs=[
            pl.BlockSpec((COUT, TILE), lambda i: (0, i)),
            pl.BlockSpec((1, COUT, 1), lambda i: (i, 0, 0)),
            pl.BlockSpec((1, COUT, 1), lambda i: (i, 0, 0)),
        ],
        out_shape=[
            jax.ShapeDtypeStruct((COUT, N_PTS), jnp.float32),
            jax.ShapeDtypeStruct((NTILE, COUT, 1), jnp.float32),
            jax.ShapeDtypeStruct((NTILE, COUT, 1), jnp.float32),
        ],
    )(qpts_t, pts1_n, pts1_t, a_t, b1p, p['l2_w'].T, p['l2_b'][:, None],
      p['l3_w'].T, p['l3_b'][:, None], gmat)


def _epi_kernel(yt_ref, s1_ref, s2_ref, bg_ref, bb_ref, fw_ref, fb_ref,
                pts_ref, c1_ref, cb1_ref, g1_ref, be1_ref,
                c2_ref, cb2_ref, g2_ref, be2_ref,
                c3_ref, cb3_ref, g3_ref, be3_ref,
                f1_ref, fb1_ref, f2_ref, fb2_ref, f3_ref, fb3_ref,
                xo_ref, roof_ref):
    n = jnp.float32(N_PTS)
    m = jnp.sum(s1_ref[...], axis=0) / n
    v = jnp.sum(s2_ref[...], axis=0) / n - m * m
    sc = bg_ref[...] / jnp.sqrt(v + 1e-5)
    sh = bb_ref[...] - m * sc
    x1d = jnp.maximum(yt_ref[...] * sc + sh, 0.0)
    xo = jnp.dot(fw_ref[...], x1d,
                 precision=jax.lax.Precision.HIGHEST, preferred_element_type=jnp.float32) + fb_ref[...]
    xo_ref[...] = xo

    m4 = jnp.max(xo, axis=0, keepdims=True)
    h0 = xo[0:1, :] >= m4
    h1 = jnp.logical_and(xo[1:2, :] >= m4, jnp.logical_not(h0))
    h01 = jnp.logical_or(h0, h1)
    h2 = jnp.logical_and(xo[2:3, :] >= m4, jnp.logical_not(h01))
    h3 = jnp.logical_not(jnp.logical_or(h01, h2))
    cmasks = [h1, h2, h3]

    rf = jnp.concatenate([x1d, pts_ref[...]], axis=0)
    lane_f = jax.lax.broadcasted_iota(
        jnp.int32, (1, N_PTS), 1).astype(jnp.float32)
    sub32 = jax.lax.broadcasted_iota(
        jnp.int32, (32, 32), 0).astype(jnp.float32)
    lane32 = jax.lax.broadcasted_iota(
        jnp.int32, (1, 32), 1).astype(jnp.float32)
    subN = jax.lax.broadcasted_iota(
        jnp.int32, (N_PTS, 32), 0).astype(jnp.float32)

    for c in range(3):
        msk = cmasks[c]
        cnt = jnp.sum(msk.astype(jnp.float32), axis=1, keepdims=True)
        vals = jnp.where(msk, lane_f, n)
        v2 = jnp.concatenate(
            [vals[:, 128 * r:128 * (r + 1)] for r in range(N_PTS // 128)],
            axis=0)
        sels = []
        for _ in range(32):
            mn = jnp.min(jnp.min(v2, axis=0, keepdims=True),
                         axis=1, keepdims=True)
            sels.append(mn)
            v2 = jnp.where(v2 == mn, n, v2)
        sel = jnp.concatenate(sels, axis=1)
        cm = jnp.maximum(jnp.minimum(cnt, 32.0), 1.0)
        jm = lane32 - jnp.floor(lane32 / cm) * cm
        perm = (sub32 == jm).astype(jnp.float32)
        self_ = jnp.dot(sel, perm,
                        precision=jax.lax.Precision.HIGHEST, preferred_element_type=jnp.float32)
        self_ = jnp.minimum(self_, n - 1.0)
        oht = (subN == self_).astype(jnp.float32)
        g = jnp.dot(rf, oht,
                    precision=jax.lax.Precision.HIGHEST, preferred_element_type=jnp.float32)
        g = jnp.where(cnt > 0.0, g, 0.0)

        def bn(o, gg, bb):
            mm = jnp.mean(o, axis=1, keepdims=True)
            vv = jnp.mean((o - mm) ** 2, axis=1, keepdims=True)
            return (o - mm) / jnp.sqrt(vv + 1e-5) * gg + bb

        o = jnp.dot(c1_ref[c], g,
                    precision=jax.lax.Precision.HIGHEST, preferred_element_type=jnp.float32) + cb1_ref[c]
        o = jnp.maximum(bn(o, g1_ref[c], be1_ref[c]), 0.0)
        o = jnp.dot(c2_ref[c], o,
                    precision=jax.lax.Precision.HIGHEST, preferred_element_type=jnp.float32) + cb2_ref[c]
        o = jnp.maximum(bn(o, g2_ref[c], be2_ref[c]), 0.0)
        o = jnp.dot(c3_ref[c], o,
                    precision=jax.lax.Precision.HIGHEST, preferred_element_type=jnp.float32) + cb3_ref[c]
        o = jnp.maximum(bn(o, g3_ref[c], be3_ref[c]), 0.0)
        f = jnp.mean(o, axis=1, keepdims=True)
        f = jnp.maximum(
            jnp.dot(f1_ref[c], f, precision=jax.lax.Precision.HIGHEST, preferred_element_type=jnp.float32)
            + fb1_ref[c], 0.0)
        f = jnp.maximum(
            jnp.dot(f2_ref[c], f, precision=jax.lax.Precision.HIGHEST, preferred_element_type=jnp.float32)
            + fb2_ref[c], 0.0)
        f = jnp.dot(f3_ref[c], f,
                    precision=jax.lax.Precision.HIGHEST, preferred_element_type=jnp.float32) + fb3_ref[c]
        roof_ref[c] = f


def _run_epilogue(yt, ys1, ys2, bng, bnb, fcwt, fcb, pts_t, pn):
    full = lambda a: pl.BlockSpec(a.shape, lambda: (0,) * a.ndim)
    ins = [yt, ys1, ys2, bng, bnb, fcwt, fcb, pts_t,
           pn['c1'], pn['cb1'], pn['g1'], pn['be1'],
           pn['c2'], pn['cb2'], pn['g2'], pn['be2'],
           pn['c3'], pn['cb3'], pn['g3'], pn['be3'],
           pn['f1t'], pn['fb1'], pn['f2t'], pn['fb2'],
           pn['f3t'], pn['fb3']]
    return pl.pallas_call(
        _epi_kernel,
        in_specs=[full(a) for a in ins],
        out_specs=[
            pl.BlockSpec((4, N_PTS), lambda: (0, 0)),
            pl.BlockSpec((3, 4, 1), lambda: (0, 0, 0)),
        ],
        out_shape=[
            jax.ShapeDtypeStruct((4, N_PTS), jnp.float32),
            jax.ShapeDtypeStruct((3, 4, 1), jnp.float32),
        ],
    )(*ins)


def _enc_ptconv(qn, srcT, srcN, srcF, K, pp):
    a_n, b1r, l2, b2r, l3, b3r, w3 = pp
    M = qn.shape[0]
    S = srcT.shape[1]
    qx = qn[:, 0:1]
    qy = qn[:, 1:2]
    qz = qn[:, 2:3]
    dx = srcT[0:1, :] - qx
    dy = srcT[1:2, :] - qy
    dz = srcT[2:3, :] - qz
    d2 = dx * dx + dy * dy + dz * dz
    srcA = jnp.dot(srcN, a_n, precision=jax.lax.Precision.HIGHEST, preferred_element_type=jnp.float32)
    qA = jnp.dot(qn, a_n, precision=jax.lax.Precision.HIGHEST, preferred_element_type=jnp.float32)
    iota = jax.lax.broadcasted_iota(jnp.int32, (M, S), 1)
    d2w = d2
    nrm_acc = None
    relas = []
    fss = []
    for _ in range(K):
        idx = jnp.argmin(d2w, axis=1, keepdims=True)
        msk = iota == idx
        oh = msk.astype(jnp.float32)
        n2 = jnp.min(d2w, axis=1, keepdims=True)
        relas.append(jnp.dot(oh, srcA,
                             precision=jax.lax.Precision.HIGHEST, preferred_element_type=jnp.float32) - qA)
        fss.append(jnp.dot(oh, srcF,
                           precision=jax.lax.Precision.HIGHEST, preferred_element_type=jnp.float32))
        nrm = jnp.sqrt(n2 + 1e-9)
        nrm_acc = nrm if nrm_acc is None else nrm_acc + nrm
        d2w = jnp.where(msk, jnp.inf, d2w)
    inv = 1.0 / (nrm_acc / K + 1e-6)
    out = jnp.zeros((M, COUT), dtype=jnp.float32)
    for k in range(K):
        h = jnp.maximum(relas[k] * inv + b1r, 0.0)
        h = jnp.maximum(
            jnp.dot(h, l2, precision=jax.lax.Precision.HIGHEST, preferred_element_type=jnp.float32) + b2r, 0.0)
        h3 = jnp.dot(h, l3, precision=jax.lax.Precision.HIGHEST, preferred_element_type=jnp.float32) + b3r
        tk = jnp.dot(fss[k], w3,
                     precision=jax.lax.Precision.HIGHEST, preferred_element_type=jnp.float32)
        for c in range(NC):
            out = out + h3[:, c:c + 1] * tk[:, COUT * c:COUT * (c + 1)]
    return out / K


def _enc_bn(x, g, b):
    m = jnp.mean(x, axis=0, keepdims=True)
    v = jnp.mean((x - m) ** 2, axis=0, keepdims=True)
    return jnp.maximum((x - m) / jnp.sqrt(v + 1e-5) * g + b, 0.0)


def _enc_kernel(x_ref, iptsT_ref, iptsN_ref, q1n_ref, q1t_ref,
                q2n_ref, q2t_ref, q3n_ref, q3t_ref, bn_ref,
                p1_0, p1_1, p1_2, p1_3, p1_4, p1_5, p1_6,
                p2_0, p2_1, p2_2, p2_3, p2_4, p2_5, p2_6,
                p3_0, p3_1, p3_2, p3_3, p3_4, p3_5, p3_6,
                p4_0, p4_1, p4_2, p4_3, p4_4, p4_5, p4_6,
                p5_0, p5_1, p5_2, p5_3, p5_4, p5_5, p5_6,
                out_ref):
    rd = lambda *refs: [r[...] for r in refs]
    bn = bn_ref[...]
    x1 = _enc_ptconv(q1n_ref[...], iptsT_ref[...], iptsN_ref[...],
                     x_ref[...], 8, rd(p1_0, p1_1, p1_2, p1_3, p1_4,
                                       p1_5, p1_6))
    x1 = _enc_bn(x1, bn[0:1, :], bn[1:2, :])
    x2 = _enc_ptconv(q2n_ref[...], q1t_ref[...], q1n_ref[...],
                     x1, 8, rd(p2_0, p2_1, p2_2, p2_3, p2_4, p2_5, p2_6))
    x2 = _enc_bn(x2, bn[2:3, :], bn[3:4, :])
    x3 = _enc_ptconv(q3n_ref[...], q2t_ref[...], q2n_ref[...],
                     x2, 4, rd(p3_0, p3_1, p3_2, p3_3, p3_4, p3_5, p3_6))
    x3 = _enc_bn(x3, bn[4:5, :], bn[5:6, :])
    x3d = _enc_ptconv(q2n_ref[...], q3t_ref[...], q3n_ref[...],
                      x3, 4, rd(p4_0, p4_1, p4_2, p4_3, p4_4, p4_5, p4_6))
    x3d = _enc_bn(x3d, bn[6:7, :], bn[7:8, :])
    x3d = jnp.concatenate([x3d, x2], axis=1)
    x2d = _enc_ptconv(q1n_ref[...], q2t_ref[...], q2n_ref[...],
                      x3d, 4, rd(p5_0, p5_1, p5_2, p5_3, p5_4, p5_5, p5_6))
    x2d = _enc_bn(x2d, bn[8:9, :], bn[9:10, :])
    out_ref[...] = jnp.concatenate([x2d, x1], axis=1)


def _enc_params(p, cin):
    l1w = p['l1_w']
    a_n = l1w.reshape(NC, DIM, 2 * NC).sum(axis=0)
    b1r = p['l1_b'][None, :] - p['centers'].reshape(1, NC * DIM) @ l1w
    w3 = jnp.transpose(p['weight'].reshape(NC, cin, COUT),
                       (1, 0, 2)).reshape(cin, NC * COUT)
    return [a_n, b1r, p['l2_w'], p['l2_b'][None, :],
            p['l3_w'], p['l3_b'][None, :], w3]


def _run_encoder(x0, ipts, params):
    q1 = ipts[::128]
    q2 = q1[::4]
    q3 = q2[::2]
    bn = jnp.stack([params['bn1_g'], params['bn1_b'],
                    params['bn2_g'], params['bn2_b'],
                    params['bn3_g'], params['bn3_b'],
                    params['bn3d_g'], params['bn3d_b'],
                    params['bn2d_g'], params['bn2d_b']])
    ins = [x0, jnp.transpose(ipts), ipts, q1, jnp.transpose(q1),
           q2, jnp.transpose(q2), q3, jnp.transpose(q3), bn]
    ins += _enc_params(params['cv1'], IN_CH)
    ins += _enc_params(params['cv2'], COUT)
    ins += _enc_params(params['cv3'], COUT)
    ins += _enc_params(params['cv3d'], COUT)
    ins += _enc_params(params['cv2d'], CIN)
    full = lambda a: pl.BlockSpec(a.shape, lambda: (0,) * a.ndim)
    out = pl.pallas_call(
        _enc_kernel,
        in_specs=[full(a) for a in ins],
        out_specs=pl.BlockSpec((NSRC, CIN), lambda: (0, 0)),
        out_shape=jax.ShapeDtypeStruct((NSRC, CIN), jnp.float32),
        compiler_params=pltpu.CompilerParams(
            vmem_limit_bytes=100 * 1024 * 1024),
    )(*ins)
    return out, q1


def _stack_pnets(pnets):
    st = lambda nm: jnp.stack([p[nm] for p in pnets])
    col = lambda nm: st(nm)[:, :, None]
    return {
        'c1': st('c1_w'),
        'cb1': col('c1_b'),
        'g1': col('bn1_g'), 'be1': col('bn1_b'),
        'c2': st('c2_w'),
        'cb2': col('c2_b'),
        'g2': col('bn2_g'), 'be2': col('bn2_b'),
        'c3': st('c3_w'),
        'cb3': col('c3_b'),
        'g3': col('bn3_g'), 'be3': col('bn3_b'),
        'f1t': jnp.transpose(st('f1_w'), (0, 2, 1)),
        'fb1': col('f1_b'),
        'f2t': jnp.transpose(st('f2_w'), (0, 2, 1)),
        'fb2': col('f2_b'),
        'f3t': jnp.transpose(st('f3_w'), (0, 2, 1)),
        'fb3': col('f3_b'),
    }


def _small_ptconv(xb, pb, K, q, p):
    d2 = jnp.sum((q[:, None, :] - pb[None, :, :]) ** 2, axis=-1)
    _, nbr = jax.lax.top_k(-d2, K)
    pts_n = pb[nbr]
    rel = pts_n - q[:, None, :]
    nrm = jnp.sqrt(jnp.sum(rel ** 2, axis=-1) + 1e-9)
    rad = jnp.mean(nrm, axis=1, keepdims=True) + 1e-6
    rel = rel / rad[:, :, None]
    dc = rel[:, :, None, :] - p['centers'][None, None, :, :]
    M = dc.shape[0]
    h = dc.reshape(M, K, NC * DIM)
    h = jax.nn.relu(h @ p['l1_w'] + p['l1_b'])
    h = jax.nn.relu(h @ p['l2_w'] + p['l2_b'])
    h = h @ p['l3_w'] + p['l3_b']
    fs = xb[nbr]
    feat = jnp.einsum('mkc,mki->mci', h, fs)
    feat = feat.reshape(M, -1) @ p['weight'] / K
    return feat


def _bn_small(x, g, b):
    m = jnp.mean(x, axis=0)
    v = jnp.var(x, axis=0)
    return jax.nn.relu((x - m) / jnp.sqrt(v + 1e-5) * g + b)


def _jax_encoder(x0, ipts, params):
    q1 = ipts[::128]
    q2 = q1[::4]
    q3 = q2[::2]
    x1 = _bn_small(_small_ptconv(x0, ipts, 8, q1, params['cv1']),
                   params['bn1_g'], params['bn1_b'])
    x2 = _bn_small(_small_ptconv(x1, q1, 8, q2, params['cv2']),
                   params['bn2_g'], params['bn2_b'])
    x3 = _bn_small(_small_ptconv(x2, q2, 4, q3, params['cv3']),
                   params['bn3_g'], params['bn3_b'])
    x3d = _bn_small(_small_ptconv(x3, q3, 4, q2, params['cv3d']),
                    params['bn3d_g'], params['bn3d_b'])
    x3d = jnp.concatenate([x3d, x2], axis=1)
    x2d = _bn_small(_small_ptconv(x3d, q2, 4, q1, params['cv2d']),
                    params['bn2d_g'], params['bn2d_b'])
    return jnp.concatenate([x2d, x1], axis=1), q1


def kernel(x, input_pts, params):
    x2d, pts1 = _jax_encoder(x[0], input_pts[0], params)

    qpts_t = jnp.transpose(input_pts[0])
    pts1_t = jnp.transpose(pts1)
    yt, ys1, ys2 = _run_cv1d(qpts_t, pts1, pts1_t,
                             jnp.transpose(x2d), params['cv1d'])

    xo_t, roof3 = _run_epilogue(
        yt, ys1, ys2, params['bn1d_g'][:, None], params['bn1d_b'][:, None],
        jnp.transpose(params['fcout_w']), params['fcout_b'][:, None],
        qpts_t, _stack_pnets(params['pnets']))
    xout = jnp.transpose(xo_t)[None]
    roof = roof3[:, :, 0][None]
    return xout, roof
